# Initial kernel scaffold; baseline (speedup 1.0000x reference)
import jax
import jax.numpy as jnp
from jax.experimental import pallas as pl


def kernel(x, adj, W0, b0, g0, beta0, W1, b1, g1, beta1):
    raise NotImplementedError("write your pallas kernel here")


# speedup vs baseline: 1.1273x; 1.1273x over previous
import functools

import jax
import jax.numpy as jnp
from jax.experimental import pallas as pl
from jax.experimental.pallas import tpu as pltpu


def _bf(v):
    return v.astype(jnp.bfloat16)


def _layer_mm_body(adj_ref, h_ref, w_ref, b_ref, t_ref):
    i = pl.program_id(0)
    r, n = adj_ref.shape
    rows = jax.lax.broadcasted_iota(jnp.int32, (r, n), 0)
    cols = jax.lax.broadcasted_iota(jnp.int32, (r, n), 1)
    a = adj_ref[...] + jnp.where(cols == rows + i * r, 1.0, 0.0)
    acc = jnp.dot(_bf(a), _bf(h_ref[...]),
                  preferred_element_type=jnp.float32)
    t_ref[...] = jnp.dot(_bf(acc), _bf(w_ref[...]),
                         preferred_element_type=jnp.float32) + b_ref[...]


def _bn_lrelu_body(t_ref, g_ref, beta_ref, o_ref, *, eps, slope):
    t = t_ref[...]
    m = jnp.mean(t, axis=0, keepdims=True)
    c = t - m
    v = jnp.mean(c * c, axis=0, keepdims=True)
    y = c * jax.lax.rsqrt(v + eps) * g_ref[...] + beta_ref[...]
    o_ref[...] = jnp.where(y >= 0, y, slope * y)


def _layer_mm(adj, h, w, b, row_block):
    n, d = h.shape
    nb = n // row_block
    return pl.pallas_call(
        _layer_mm_body,
        grid=(nb,),
        in_specs=[
            pl.BlockSpec((row_block, n), lambda i: (i, 0)),
            pl.BlockSpec((n, d), lambda i: (0, 0)),
            pl.BlockSpec((d, d), lambda i: (0, 0)),
            pl.BlockSpec((1, d), lambda i: (0, 0)),
        ],
        out_specs=pl.BlockSpec((row_block, d), lambda i: (i, 0)),
        out_shape=jax.ShapeDtypeStruct((n, d), jnp.float32),
        compiler_params=pltpu.CompilerParams(
            dimension_semantics=("arbitrary",),
        ),
    )(adj, h, w, b)


def _bn_lrelu(t, g, beta):
    n, d = t.shape
    body = functools.partial(_bn_lrelu_body, eps=1e-5, slope=0.01)
    return pl.pallas_call(
        body,
        in_specs=[
            pl.BlockSpec((n, d), lambda: (0, 0)),
            pl.BlockSpec((1, d), lambda: (0, 0)),
            pl.BlockSpec((1, d), lambda: (0, 0)),
        ],
        out_specs=pl.BlockSpec((n, d), lambda: (0, 0)),
        out_shape=jax.ShapeDtypeStruct((n, d), jnp.float32),
    )(t, g, beta)


def kernel(x, adj, W0, b0, g0, beta0, W1, b1, g1, beta1):
    n = adj.shape[0]
    row_block = 400 if n % 400 == 0 else n
    h = x
    for (w, b, g, beta) in ((W0, b0, g0, beta0), (W1, b1, g1, beta1)):
        t = _layer_mm(adj, h, w, b.reshape(1, -1), row_block)
        h = _bn_lrelu(t, g.reshape(1, -1), beta.reshape(1, -1))
    return h
